# Initial kernel scaffold; baseline (speedup 1.0000x reference)
import jax
import jax.numpy as jnp
from jax.experimental import pallas as pl


def kernel(nucleotides, positions, base_table, pos_table, W, b):
    raise NotImplementedError("write your pallas kernel here")


# speedup vs baseline: 2.0689x; 2.0689x over previous
import functools

import jax
import jax.numpy as jnp
from jax import lax
from jax.experimental import pallas as pl
from jax.experimental.pallas import tpu as pltpu
from jax.experimental.pallas import tpu_sc as plsc

_B, _S, _D = 4096, 200, 32
_E = _B * _S
_NC, _NS = 2, 16
_NW = _NC * _NS
_PER_W = _E // _NW
_CHUNK = 1024
_NCHUNK = _PER_W // _CHUNK
_GWIN = 128


def _table_body(base_ref, pos_ref, w_ref, b_ref, out_ref):
    cid = lax.broadcasted_iota(jnp.int32, (16, 1), 0)
    nid = jnp.minimum(cid // 3, 4)
    pid = cid - (cid // 3) * 3
    eb = jnp.zeros((16, _D), jnp.float32)
    for r in range(5):
        eb = eb + jnp.where(nid == r, 1.0, 0.0) * base_ref[r : r + 1, :]
    ep = jnp.zeros((16, _D), jnp.float32)
    for r in range(3):
        ep = ep + jnp.where(pid == r, 1.0, 0.0) * pos_ref[r : r + 1, :]
    e = jnp.concatenate([eb, ep], axis=1)
    out_ref[...] = (
        jnp.dot(e, w_ref[...], preferred_element_type=jnp.float32) + b_ref[...]
    )


def _build_table(base_table, pos_table, W, b):
    return pl.pallas_call(
        _table_body,
        out_shape=jax.ShapeDtypeStruct((16, _D), jnp.float32),
    )(base_table, pos_table, W, b.reshape(1, _D))


def _sc_body(tbl_hbm, n_hbm, p_hbm, out_hbm, n_v, p_v, c_v, rows_v, isem, gsem):
    wid = lax.axis_index("s") * _NC + lax.axis_index("c")
    base = wid * _PER_W

    @pl.loop(0, _NCHUNK)
    def _chunk(g):
        off = base + g * _CHUNK
        cp_n = pltpu.async_copy(n_hbm.at[pl.ds(off, _CHUNK)], n_v, isem)
        cp_p = pltpu.async_copy(p_hbm.at[pl.ds(off, _CHUNK)], p_v, isem)
        cp_n.wait()
        cp_p.wait()

        @pl.loop(0, _CHUNK, step=16)
        def _cvt(i):
            s = pl.ds(i, 16)
            c_v[s] = n_v[s] * 3 + p_v[s]

        gathers = [
            pltpu.async_copy(
                tbl_hbm.at[c_v.at[pl.ds(j * _GWIN, _GWIN)]],
                rows_v.at[pl.ds(j * _GWIN, _GWIN)],
                gsem,
            )
            for j in range(_CHUNK // _GWIN)
        ]
        for cp in gathers:
            cp.wait()
        pltpu.sync_copy(rows_v, out_hbm.at[pl.ds(off, _CHUNK)])


@jax.jit
def _sc_lookup(tbl, n_flat, p_flat):
    mesh = plsc.VectorSubcoreMesh(core_axis_name="c", subcore_axis_name="s")
    run = pl.kernel(
        _sc_body,
        out_type=jax.ShapeDtypeStruct((_E, _D), jnp.float32),
        mesh=mesh,
        scratch_types=[
            pltpu.VMEM((_CHUNK,), jnp.int32),
            pltpu.VMEM((_CHUNK,), jnp.int32),
            pltpu.VMEM((_CHUNK,), jnp.int32),
            pltpu.VMEM((_CHUNK, _D), jnp.float32),
            pltpu.SemaphoreType.DMA,
            pltpu.SemaphoreType.DMA,
        ],
        compiler_params=pltpu.CompilerParams(use_tc_tiling_on_sc=False),
    )
    return run(tbl, n_flat, p_flat)


def kernel(nucleotides, positions, base_table, pos_table, W, b):
    tbl = _build_table(base_table, pos_table, W, b)
    n_flat = nucleotides.reshape(_E).astype(jnp.int32)
    p_flat = positions.reshape(_E).astype(jnp.int32)
    out = _sc_lookup(tbl, n_flat, p_flat)
    return out.reshape(_B, _S, _D)


# speedup vs baseline: 2.0779x; 1.0044x over previous
import functools

import jax
import jax.numpy as jnp
from jax import lax
from jax.experimental import pallas as pl
from jax.experimental.pallas import tpu as pltpu
from jax.experimental.pallas import tpu_sc as plsc

_B, _S, _D = 4096, 200, 32
_E = _B * _S
_NC, _NS = 2, 16
_NW = _NC * _NS
_PER_W = _E // _NW
_CHUNK = 1280
_NCHUNK = _PER_W // _CHUNK
_GWIN = 128
_NG = _CHUNK // _GWIN


def _table_body(base_ref, pos_ref, w_ref, b_ref, out_ref):
    cid = lax.broadcasted_iota(jnp.int32, (16, 1), 0)
    nid = jnp.minimum(cid // 3, 4)
    pid = cid - (cid // 3) * 3
    eb = jnp.zeros((16, _D), jnp.float32)
    for r in range(5):
        eb = eb + jnp.where(nid == r, 1.0, 0.0) * base_ref[r : r + 1, :]
    ep = jnp.zeros((16, _D), jnp.float32)
    for r in range(3):
        ep = ep + jnp.where(pid == r, 1.0, 0.0) * pos_ref[r : r + 1, :]
    e = jnp.concatenate([eb, ep], axis=1)
    out_ref[...] = (
        jnp.dot(e, w_ref[...], preferred_element_type=jnp.float32) + b_ref[...]
    )


def _build_table(base_table, pos_table, W, b):
    return pl.pallas_call(
        _table_body,
        out_shape=jax.ShapeDtypeStruct((16, _D), jnp.float32),
    )(base_table, pos_table, W, b.reshape(1, _D))


def _sc_body(tbl_hbm, n_hbm, p_hbm, out_hbm, *scratch):
    n_v = scratch[0:2]
    p_v = scratch[2:4]
    c_v = scratch[4:6]
    rows_v = scratch[6:8]
    isem = scratch[8:10]
    gsem = scratch[10:12]
    wsem = scratch[12:14]
    wid = lax.axis_index("s") * _NC + lax.axis_index("c")
    base = wid * _PER_W

    def idx_start(g, b):
        off = base + g * _CHUNK
        pltpu.async_copy(n_hbm.at[pl.ds(off, _CHUNK)], n_v[b], isem[b])
        pltpu.async_copy(p_hbm.at[pl.ds(off, _CHUNK)], p_v[b], isem[b])

    def idx_wait(b):
        pltpu.make_async_copy(n_hbm.at[pl.ds(0, _CHUNK)], n_v[b], isem[b]).wait()
        pltpu.make_async_copy(p_hbm.at[pl.ds(0, _CHUNK)], p_v[b], isem[b]).wait()

    def compute_c(b):
        nb, pb, cb = n_v[b], p_v[b], c_v[b]

        @pl.loop(0, _CHUNK, step=16)
        def _cvt(i):
            s = pl.ds(i, 16)
            cb[s] = nb[s] * 3 + pb[s]

    def gathers_start(b):
        for j in range(_NG):
            s = pl.ds(j * _GWIN, _GWIN)
            pltpu.async_copy(tbl_hbm.at[c_v[b].at[s]], rows_v[b].at[s], gsem[b])

    def gathers_wait(b):
        for j in range(_NG):
            s = pl.ds(j * _GWIN, _GWIN)
            pltpu.make_async_copy(
                tbl_hbm.at[c_v[b].at[s]], rows_v[b].at[s], gsem[b]
            ).wait()

    def write_start(g, b):
        off = base + g * _CHUNK
        pltpu.async_copy(rows_v[b], out_hbm.at[pl.ds(off, _CHUNK)], wsem[b])

    def write_wait(b):
        pltpu.make_async_copy(
            rows_v[b], out_hbm.at[pl.ds(0, _CHUNK)], wsem[b]
        ).wait()

    idx_start(0, 0)
    idx_start(1, 1)

    @pl.loop(0, _NCHUNK // 2)
    def _pair(gg):
        for b in range(2):
            g = gg * 2 + b
            idx_wait(b)

            @pl.when(g >= 2)
            def _():
                write_wait(b)

            compute_c(b)
            gathers_start(b)

            @pl.when(g + 2 < _NCHUNK)
            def _():
                idx_start(g + 2, b)

            @pl.when(g >= 1)
            def _():
                gathers_wait(b ^ 1)
                write_start(g - 1, b ^ 1)

    last = _NCHUNK - 1
    gathers_wait(last & 1)
    write_start(last, last & 1)
    write_wait(last & 1)
    write_wait((last & 1) ^ 1)


@jax.jit
def _sc_lookup(tbl, n_flat, p_flat):
    mesh = plsc.VectorSubcoreMesh(core_axis_name="c", subcore_axis_name="s")
    run = pl.kernel(
        _sc_body,
        out_type=jax.ShapeDtypeStruct((_E, _D), jnp.float32),
        mesh=mesh,
        scratch_types=(
            [pltpu.VMEM((_CHUNK,), jnp.int32) for _ in range(2)]
            + [pltpu.VMEM((_CHUNK,), jnp.int32) for _ in range(2)]
            + [pltpu.VMEM((_CHUNK,), jnp.int32) for _ in range(2)]
            + [pltpu.VMEM((_CHUNK, _D), jnp.float32) for _ in range(2)]
            + [pltpu.SemaphoreType.DMA for _ in range(6)]
        ),
        compiler_params=pltpu.CompilerParams(use_tc_tiling_on_sc=False),
    )
    return run(tbl, n_flat, p_flat)


def kernel(nucleotides, positions, base_table, pos_table, W, b):
    tbl = _build_table(base_table, pos_table, W, b)
    n_flat = nucleotides.reshape(_E).astype(jnp.int32)
    p_flat = positions.reshape(_E).astype(jnp.int32)
    out = _sc_lookup(tbl, n_flat, p_flat)
    return out.reshape(_B, _S, _D)


# speedup vs baseline: 9.9471x; 4.7870x over previous
import dataclasses

import jax
import jax.numpy as jnp
from jax import lax
from jax.experimental import pallas as pl
from jax.experimental.pallas import tpu as pltpu
from jax.experimental.pallas import tpu_sc as plsc

_B, _S, _D = 4096, 200, 32
_E = _B * _S
_Q = _E // 4
_NC, _NS = 2, 16
_NW = _NC * _NS
_PER_W = _E // _NW
_CHUNK = 1600
_QCHUNK = _CHUNK // 4
_NCHUNK = _PER_W // _CHUNK
_GWIN = 80
_NG = _QCHUNK // _GWIN
_TROWS = 16 * 16 * 16 * 16


def _table_body(base_ref, pos_ref, w_ref, b_ref, out_ref):
    cid = lax.broadcasted_iota(jnp.int32, (16, 1), 0)
    nid = jnp.minimum(cid // 3, 4)
    pid = cid - (cid // 3) * 3
    eb = jnp.zeros((16, _D), jnp.float32)
    for r in range(5):
        eb = eb + jnp.where(nid == r, 1.0, 0.0) * base_ref[r : r + 1, :]
    ep = jnp.zeros((16, _D), jnp.float32)
    for r in range(3):
        ep = ep + jnp.where(pid == r, 1.0, 0.0) * pos_ref[r : r + 1, :]
    e = jnp.concatenate([eb, ep], axis=1)
    out_ref[...] = (
        jnp.dot(e, w_ref[...], preferred_element_type=jnp.float32) + b_ref[...]
    )


def _build_table(base_table, pos_table, W, b):
    return pl.pallas_call(
        _table_body,
        out_shape=jax.ShapeDtypeStruct((16, _D), jnp.float32),
    )(base_table, pos_table, W, b.reshape(1, _D))


_QBLK = 2048


def _quad_body(fused_ref, out_ref):
    i = pl.program_id(0)
    q = i * _QBLK + lax.broadcasted_iota(jnp.int32, (_QBLK, 1), 0)
    lane = lax.broadcasted_iota(jnp.int32, (1, 16), 1)
    fused = fused_ref[...]
    parts = []
    for k in range(4):
        ck = (q >> (12 - 4 * k)) & 15
        onehot = jnp.where(ck == lane, 1.0, 0.0)
        parts.append(
            jnp.dot(onehot, fused, preferred_element_type=jnp.float32)
        )
    out_ref[...] = jnp.concatenate(parts, axis=1)


def _build_quad_table(fused):
    return pl.pallas_call(
        _quad_body,
        grid=(_TROWS // _QBLK,),
        in_specs=[pl.BlockSpec((16, _D), lambda i: (0, 0))],
        out_specs=pl.BlockSpec((_QBLK, 4 * _D), lambda i: (i, 0)),
        out_shape=jax.ShapeDtypeStruct((_TROWS, 4 * _D), jnp.float32),
    )(fused)


def _sc_body(tbl_hbm, n_hbm, p_hbm, out_hbm, *scratch):
    n_v = scratch[0:2]
    p_v = scratch[2:4]
    q_v = scratch[4:6]
    rows_v = scratch[6:8]
    isem = scratch[8:10]
    gsem = scratch[10:12]
    wsem = scratch[12:14]
    wid = lax.axis_index("s") * _NC + lax.axis_index("c")
    ebase = wid * _PER_W
    qbase = wid * (_PER_W // 4)

    def idx_start(g, b):
        off = ebase + g * _CHUNK
        pltpu.async_copy(n_hbm.at[pl.ds(off, _CHUNK)], n_v[b], isem[b])
        pltpu.async_copy(p_hbm.at[pl.ds(off, _CHUNK)], p_v[b], isem[b])

    def idx_wait(b):
        pltpu.make_async_copy(n_hbm.at[pl.ds(0, _CHUNK)], n_v[b], isem[b]).wait()
        pltpu.make_async_copy(p_hbm.at[pl.ds(0, _CHUNK)], p_v[b], isem[b]).wait()

    def compute_q(b):
        nb, pb, qb = n_v[b], p_v[b], q_v[b]
        i4 = lax.iota(jnp.int32, 16) * 4

        @pl.loop(0, _QCHUNK, step=16)
        def _pack(t):
            e0 = t * 4 + i4
            acc = None
            for k in range(4):
                ek = e0 + k
                ck = plsc.load_gather(nb, [ek]) * 3 + plsc.load_gather(pb, [ek])
                acc = ck if acc is None else acc * 16 + ck
            qb[pl.ds(t, 16)] = acc

    def gathers_start(b):
        for j in range(_NG):
            s = pl.ds(j * _GWIN, _GWIN)
            pltpu.async_copy(tbl_hbm.at[q_v[b].at[s]], rows_v[b].at[s], gsem[b])

    def gathers_wait(b):
        for j in range(_NG):
            s = pl.ds(j * _GWIN, _GWIN)
            pltpu.make_async_copy(
                tbl_hbm.at[q_v[b].at[s]], rows_v[b].at[s], gsem[b]
            ).wait()

    def write_start(g, b):
        off = qbase + g * _QCHUNK
        pltpu.async_copy(rows_v[b], out_hbm.at[pl.ds(off, _QCHUNK)], wsem[b])

    def write_wait(b):
        pltpu.make_async_copy(
            rows_v[b], out_hbm.at[pl.ds(0, _QCHUNK)], wsem[b]
        ).wait()

    idx_start(0, 0)
    idx_start(1, 1)

    @pl.loop(0, _NCHUNK // 2)
    def _pair(gg):
        for b in range(2):
            g = gg * 2 + b
            idx_wait(b)

            @pl.when(g >= 2)
            def _():
                write_wait(b)

            compute_q(b)
            gathers_start(b)

            @pl.when(g + 2 < _NCHUNK)
            def _():
                idx_start(g + 2, b)

            @pl.when(g >= 1)
            def _():
                gathers_wait(b ^ 1)
                write_start(g - 1, b ^ 1)

    last = _NCHUNK - 1
    gathers_wait(last & 1)
    write_start(last, last & 1)
    write_wait(last & 1)
    write_wait((last & 1) ^ 1)


@jax.jit
def _sc_lookup(tbl, n_flat, p_flat):
    mesh = plsc.VectorSubcoreMesh(core_axis_name="c", subcore_axis_name="s")
    run = pl.kernel(
        _sc_body,
        out_type=jax.ShapeDtypeStruct((_Q, 4 * _D), jnp.float32),
        mesh=mesh,
        scratch_types=(
            [pltpu.VMEM((_CHUNK,), jnp.int32) for _ in range(2)]
            + [pltpu.VMEM((_CHUNK,), jnp.int32) for _ in range(2)]
            + [pltpu.VMEM((_QCHUNK,), jnp.int32) for _ in range(2)]
            + [pltpu.VMEM((_QCHUNK, 4 * _D), jnp.float32) for _ in range(2)]
            + [pltpu.SemaphoreType.DMA for _ in range(6)]
        ),
        compiler_params=_sc_compiler_params(),
    )
    return run(tbl, n_flat, p_flat)


def _sc_compiler_params():
    cp = pltpu.CompilerParams()
    if "needs_layout_passes" in pltpu.CompilerParams.__dataclass_fields__:
        cp = dataclasses.replace(cp, needs_layout_passes=False)
    return cp


def kernel(nucleotides, positions, base_table, pos_table, W, b):
    fused = _build_table(base_table, pos_table, W, b)
    tbl4 = _build_quad_table(fused)
    n_flat = nucleotides.reshape(_E).astype(jnp.int32)
    p_flat = positions.reshape(_E).astype(jnp.int32)
    out = _sc_lookup(tbl4, n_flat, p_flat)
    return out.reshape(_B, _S, _D)


# speedup vs baseline: 17.4715x; 1.7564x over previous
import dataclasses

import jax
import jax.numpy as jnp
from jax import lax
from jax.experimental import pallas as pl
from jax.experimental.pallas import tpu as pltpu
from jax.experimental.pallas import tpu_sc as plsc

_B, _S, _D = 4096, 200, 32
_E = _B * _S
_Q = _E // 4
_NC, _NS = 2, 16
_NW = _NC * _NS
_PER_W = _E // _NW
_CHUNK = 1600
_QCHUNK = _CHUNK // 4
_NCHUNK = _PER_W // _CHUNK
_GWIN = 80
_NG = _QCHUNK // _GWIN
_TROWS = 16 * 16 * 16 * 16


def _table_body(base_ref, pos_ref, w_ref, b_ref, out_ref):
    cid = lax.broadcasted_iota(jnp.int32, (16, 1), 0)
    nid = jnp.minimum(cid // 3, 4)
    pid = cid - (cid // 3) * 3
    eb = jnp.zeros((16, _D), jnp.float32)
    for r in range(5):
        eb = eb + jnp.where(nid == r, 1.0, 0.0) * base_ref[r : r + 1, :]
    ep = jnp.zeros((16, _D), jnp.float32)
    for r in range(3):
        ep = ep + jnp.where(pid == r, 1.0, 0.0) * pos_ref[r : r + 1, :]
    e = jnp.concatenate([eb, ep], axis=1)
    out_ref[...] = (
        jnp.dot(e, w_ref[...], preferred_element_type=jnp.float32) + b_ref[...]
    )


def _build_table(base_table, pos_table, W, b):
    return pl.pallas_call(
        _table_body,
        out_shape=jax.ShapeDtypeStruct((16, _D), jnp.float32),
    )(base_table, pos_table, W, b.reshape(1, _D))


_QBLK = 2048


def _quad_body(fused_ref, out_ref):
    i = pl.program_id(0)
    q = i * _QBLK + lax.broadcasted_iota(jnp.int32, (_QBLK, 1), 0)
    lane = lax.broadcasted_iota(jnp.int32, (1, 16), 1)
    fused = fused_ref[...]
    parts = []
    for k in range(4):
        ck = (q >> (12 - 4 * k)) & 15
        onehot = jnp.where(ck == lane, 1.0, 0.0)
        parts.append(
            jnp.dot(onehot, fused, preferred_element_type=jnp.float32)
        )
    out_ref[...] = jnp.concatenate(parts, axis=1)


def _build_quad_table(fused):
    return pl.pallas_call(
        _quad_body,
        grid=(_TROWS // _QBLK,),
        in_specs=[pl.BlockSpec((16, _D), lambda i: (0, 0))],
        out_specs=pl.BlockSpec((_QBLK, 4 * _D), lambda i: (i, 0)),
        out_shape=jax.ShapeDtypeStruct((_TROWS, 4 * _D), jnp.float32),
    )(fused)


def _sc_body(tbl_hbm, n_hbm, p_hbm, out_hbm, *scratch):
    n_v = scratch[0:2]
    p_v = scratch[2:4]
    q_v = scratch[4:6]
    rows_v = scratch[6:8]
    isem = scratch[8:10]
    gsem = scratch[10:12]
    wsem = scratch[12:14]
    wid = lax.axis_index("s") * _NC + lax.axis_index("c")
    ebase = wid * _PER_W
    qbase = wid * (_PER_W // 4)

    def idx_start(g, b):
        off = ebase + g * _CHUNK
        pltpu.async_copy(n_hbm.at[pl.ds(off, _CHUNK)], n_v[b], isem[b])
        pltpu.async_copy(p_hbm.at[pl.ds(off, _CHUNK)], p_v[b], isem[b])

    def idx_wait(b):
        pltpu.make_async_copy(n_hbm.at[pl.ds(0, _CHUNK)], n_v[b], isem[b]).wait()
        pltpu.make_async_copy(p_hbm.at[pl.ds(0, _CHUNK)], p_v[b], isem[b]).wait()

    def compute_q(b):
        nb, pb, qb = n_v[b], p_v[b], q_v[b]
        i4 = lax.iota(jnp.int32, 16) * 4

        @pl.loop(0, _QCHUNK, step=16)
        def _pack(t):
            e0 = t * 4 + i4
            acc = None
            for k in range(4):
                ek = e0 + k
                ck = plsc.load_gather(nb, [ek]) * 3 + plsc.load_gather(pb, [ek])
                acc = ck if acc is None else acc * 16 + ck
            qb[pl.ds(t, 16)] = acc

    def gathers_start(b):
        for j in range(_NG):
            s = pl.ds(j * _GWIN, _GWIN)
            pltpu.async_copy(tbl_hbm.at[q_v[b].at[s]], rows_v[b].at[s], gsem[b])

    def gathers_wait(b):
        for j in range(_NG):
            s = pl.ds(j * _GWIN, _GWIN)
            pltpu.make_async_copy(
                tbl_hbm.at[q_v[b].at[s]], rows_v[b].at[s], gsem[b]
            ).wait()

    def write_start(g, b):
        off = qbase + g * _QCHUNK
        pltpu.async_copy(rows_v[b], out_hbm.at[pl.ds(off, _QCHUNK)], wsem[b])

    def write_wait(b):
        pltpu.make_async_copy(
            rows_v[b], out_hbm.at[pl.ds(0, _QCHUNK)], wsem[b]
        ).wait()

    idx_start(0, 0)
    idx_start(1, 1)

    @pl.loop(0, _NCHUNK // 2)
    def _pair(gg):
        for b in range(2):
            g = gg * 2 + b
            idx_wait(b)

            @pl.when(g >= 2)
            def _():
                write_wait(b)

            compute_q(b)
            gathers_start(b)

            @pl.when(g + 2 < _NCHUNK)
            def _():
                idx_start(g + 2, b)

            @pl.when(g >= 1)
            def _():
                gathers_wait(b ^ 1)
                write_start(g - 1, b ^ 1)

    last = _NCHUNK - 1
    gathers_wait(last & 1)
    write_start(last, last & 1)
    write_wait(last & 1)
    write_wait((last & 1) ^ 1)


@jax.jit
def _sc_lookup(tbl, n_flat, p_flat):
    mesh = plsc.VectorSubcoreMesh(core_axis_name="c", subcore_axis_name="s")
    run = pl.kernel(
        _sc_body,
        out_type=jax.ShapeDtypeStruct((_Q, 4 * _D), jnp.float32),
        mesh=mesh,
        scratch_types=(
            [pltpu.VMEM((_CHUNK,), jnp.int32) for _ in range(2)]
            + [pltpu.VMEM((_CHUNK,), jnp.int32) for _ in range(2)]
            + [pltpu.VMEM((_QCHUNK,), jnp.int32) for _ in range(2)]
            + [pltpu.VMEM((_QCHUNK, 4 * _D), jnp.float32) for _ in range(2)]
            + [pltpu.SemaphoreType.DMA for _ in range(6)]
        ),
        compiler_params=_sc_compiler_params(),
    )
    return run(tbl, n_flat, p_flat)


def _sc_compiler_params():
    cp = pltpu.CompilerParams()
    if "needs_layout_passes" in pltpu.CompilerParams.__dataclass_fields__:
        cp = dataclasses.replace(cp, needs_layout_passes=False)
    return cp


_TI = 1024


def _xpose_body(in_ref, out_ref):
    xt = in_ref[:, 0, 0, :].T
    for j in range(4):
        out_ref[j] = xt[j * _D : (j + 1) * _D, :]


def _to_output_layout(out_sc):
    in3 = out_sc.reshape(_B, _S // 4, 1, 128)
    out_t = pl.pallas_call(
        _xpose_body,
        grid=(_S // 4, _B // _TI),
        in_specs=[
            pl.BlockSpec((_TI, 1, 1, 128), lambda sq, ib: (ib, sq, 0, 0))
        ],
        out_specs=pl.BlockSpec((4, _D, _TI), lambda sq, ib: (sq, 0, ib)),
        out_shape=jax.ShapeDtypeStruct((_S, _D, _B), jnp.float32),
    )(in3)
    return jnp.transpose(out_t, (2, 0, 1))


def kernel(nucleotides, positions, base_table, pos_table, W, b):
    fused = _build_table(base_table, pos_table, W, b)
    tbl4 = _build_quad_table(fused)
    n_flat = nucleotides.reshape(_E).astype(jnp.int32)
    p_flat = positions.reshape(_E).astype(jnp.int32)
    out_sc = _sc_lookup(tbl4, n_flat, p_flat)
    return _to_output_layout(out_sc)


# speedup vs baseline: 18.8061x; 1.0764x over previous
import dataclasses

import jax
import jax.numpy as jnp
from jax import lax
from jax.experimental import pallas as pl
from jax.experimental.pallas import tpu as pltpu
from jax.experimental.pallas import tpu_sc as plsc

_B, _S, _D = 4096, 200, 32
_E = _B * _S
_Q = _E // 4
_NC, _NS = 2, 16
_NW = _NC * _NS
_CHUNK = 1024
_QCHUNK = _CHUNK // 4
_NCHUNK = _E // _CHUNK // _NW
_GWIN = 128
_NG = _QCHUNK // _GWIN
_TROWS = 16 * 16 * 16 * 16
_QBLK = 2048


def _quad_body(base_ref, pos_ref, w_ref, b_ref, out_ref):
    cid = lax.broadcasted_iota(jnp.int32, (16, 1), 0)
    nid = jnp.minimum(cid // 3, 4)
    pid = cid - (cid // 3) * 3
    eb = jnp.zeros((16, _D), jnp.float32)
    for r in range(5):
        eb = eb + jnp.where(nid == r, 1.0, 0.0) * base_ref[r : r + 1, :]
    ep = jnp.zeros((16, _D), jnp.float32)
    for r in range(3):
        ep = ep + jnp.where(pid == r, 1.0, 0.0) * pos_ref[r : r + 1, :]
    e = jnp.concatenate([eb, ep], axis=1)
    fused = (
        jnp.dot(e, w_ref[...], preferred_element_type=jnp.float32) + b_ref[...]
    )
    i = pl.program_id(0)
    q = i * _QBLK + lax.broadcasted_iota(jnp.int32, (_QBLK, 1), 0)
    lane = lax.broadcasted_iota(jnp.int32, (1, 16), 1)
    parts = []
    for k in range(4):
        ck = (q >> (12 - 4 * k)) & 15
        onehot = jnp.where(ck == lane, 1.0, 0.0)
        parts.append(
            jnp.dot(onehot, fused, preferred_element_type=jnp.float32)
        )
    out_ref[...] = jnp.concatenate(parts, axis=1)


def _build_quad_table(base_table, pos_table, W, b):
    return pl.pallas_call(
        _quad_body,
        grid=(_TROWS // _QBLK,),
        in_specs=[
            pl.BlockSpec((5, _D), lambda i: (0, 0)),
            pl.BlockSpec((3, _D), lambda i: (0, 0)),
            pl.BlockSpec((2 * _D, _D), lambda i: (0, 0)),
            pl.BlockSpec((1, _D), lambda i: (0, 0)),
        ],
        out_specs=pl.BlockSpec((_QBLK, 4 * _D), lambda i: (i, 0)),
        out_shape=jax.ShapeDtypeStruct((_TROWS, 4 * _D), jnp.float32),
    )(base_table, pos_table, W, b.reshape(1, _D))


def _sc_body(tbl_hbm, n_hbm, p_hbm, out_hbm, *scratch):
    n_v = scratch[0:2]
    p_v = scratch[2:4]
    q_v = scratch[4:6]
    rows_v = scratch[6:8]
    isem = scratch[8:10]
    gsem = scratch[10:12]
    wsem = scratch[12:14]
    wid = lax.axis_index("s") * _NC + lax.axis_index("c")
    ebase = wid * _NCHUNK * _CHUNK
    qbase = wid * _NCHUNK * _QCHUNK

    def idx_start(g, b):
        off = ebase + g * _CHUNK
        pltpu.async_copy(n_hbm.at[pl.ds(off, _CHUNK)], n_v[b], isem[b])
        pltpu.async_copy(p_hbm.at[pl.ds(off, _CHUNK)], p_v[b], isem[b])

    def idx_wait(b):
        pltpu.make_async_copy(n_hbm.at[pl.ds(0, _CHUNK)], n_v[b], isem[b]).wait()
        pltpu.make_async_copy(p_hbm.at[pl.ds(0, _CHUNK)], p_v[b], isem[b]).wait()

    def compute_q(b):
        nb, pb, qb = n_v[b], p_v[b], q_v[b]

        @pl.loop(0, _QCHUNK, step=16)
        def _pack(q0):
            acc = nb[pl.ds(q0, 16)] * 3 + pb[pl.ds(q0, 16)]
            for j in range(1, 4):
                s_ = pl.ds(q0 + _QCHUNK * j, 16)
                acc = acc * 16 + (nb[s_] * 3 + pb[s_])
            qb[pl.ds(q0, 16)] = acc

    def gathers_start(b):
        for j in range(_NG):
            s_ = pl.ds(j * _GWIN, _GWIN)
            pltpu.async_copy(
                tbl_hbm.at[q_v[b].at[s_]], rows_v[b].at[s_], gsem[b]
            )

    def gathers_wait(b):
        for j in range(_NG):
            s_ = pl.ds(j * _GWIN, _GWIN)
            pltpu.make_async_copy(
                tbl_hbm.at[q_v[b].at[s_]], rows_v[b].at[s_], gsem[b]
            ).wait()

    def write_start(g, b):
        off = qbase + g * _QCHUNK
        pltpu.async_copy(rows_v[b], out_hbm.at[pl.ds(off, _QCHUNK)], wsem[b])

    def write_wait(b):
        pltpu.make_async_copy(
            rows_v[b], out_hbm.at[pl.ds(0, _QCHUNK)], wsem[b]
        ).wait()

    def step(g, b):
        idx_wait(b)

        @pl.when(g >= 2)
        def _():
            write_wait(b)

        compute_q(b)
        gathers_start(b)

        @pl.when(g + 2 < _NCHUNK)
        def _():
            idx_start(g + 2, b)

        @pl.when(g >= 1)
        def _():
            gathers_wait(b ^ 1)
            write_start(g - 1, b ^ 1)

    idx_start(0, 0)
    idx_start(1, 1)

    @pl.loop(0, (_NCHUNK - 1) // 2)
    def _pair(gg):
        for b in range(2):
            step(gg * 2 + b, b)

    last = _NCHUNK - 1
    step(last, last & 1)
    gathers_wait(last & 1)
    write_start(last, last & 1)
    write_wait(last & 1)
    write_wait((last & 1) ^ 1)


def _sc_compiler_params():
    cp = pltpu.CompilerParams()
    if "needs_layout_passes" in pltpu.CompilerParams.__dataclass_fields__:
        cp = dataclasses.replace(cp, needs_layout_passes=False)
    return cp


@jax.jit
def _sc_lookup(tbl, n_flat, p_flat):
    mesh = plsc.VectorSubcoreMesh(core_axis_name="c", subcore_axis_name="s")
    run = pl.kernel(
        _sc_body,
        out_type=jax.ShapeDtypeStruct((_Q, 4 * _D), jnp.float32),
        mesh=mesh,
        scratch_types=(
            [pltpu.VMEM((_CHUNK,), jnp.int32) for _ in range(2)]
            + [pltpu.VMEM((_CHUNK,), jnp.int32) for _ in range(2)]
            + [pltpu.VMEM((_QCHUNK,), jnp.int32) for _ in range(2)]
            + [pltpu.VMEM((_QCHUNK, 4 * _D), jnp.float32) for _ in range(2)]
            + [pltpu.SemaphoreType.DMA for _ in range(6)]
        ),
        compiler_params=_sc_compiler_params(),
    )
    return run(tbl, n_flat, p_flat)


def _xpose_body(in_ref, out_ref):
    xt = in_ref[0].T
    for j in range(4):
        for k in range(4):
            out_ref[0, :, 1024 * k + 256 * j : 1024 * k + 256 * j + 256] = xt[
                32 * j : 32 * (j + 1), 256 * k : 256 * (k + 1)
            ]


def _to_output_layout(out_sc):
    in3 = out_sc.reshape(_S, _B // 4, 128)
    out_t = pl.pallas_call(
        _xpose_body,
        grid=(_S,),
        in_specs=[pl.BlockSpec((1, _B // 4, 128), lambda s: (s, 0, 0))],
        out_specs=pl.BlockSpec((1, _D, _B), lambda s: (s, 0, 0)),
        out_shape=jax.ShapeDtypeStruct((_S, _D, _B), jnp.float32),
    )(in3)
    return jnp.transpose(out_t, (2, 0, 1))


def kernel(nucleotides, positions, base_table, pos_table, W, b):
    tbl4 = _build_quad_table(base_table, pos_table, W, b)
    n_flat = jnp.transpose(nucleotides).reshape(_E).astype(jnp.int32)
    p_flat = jnp.transpose(positions).reshape(_E).astype(jnp.int32)
    out_sc = _sc_lookup(tbl4, n_flat, p_flat)
    return _to_output_layout(out_sc)


# speedup vs baseline: 26.1785x; 1.3920x over previous
import dataclasses

import jax
import jax.numpy as jnp
from jax import lax
from jax.experimental import pallas as pl
from jax.experimental.pallas import tpu as pltpu
from jax.experimental.pallas import tpu_sc as plsc

_B, _S, _D = 4096, 200, 32
_E = _B * _S
_Q = _E // 4
_NC, _NS = 2, 16
_NW = _NC * _NS
_CHUNK = 1024
_QCHUNK = _CHUNK // 4
_NCHUNK = _E // _CHUNK // _NW
_GWIN = 128
_NG = _QCHUNK // _GWIN
_TROWS = 16 * 16 * 16 * 16
_QBLK = 2048


def _quad_body(base_ref, pos_ref, w_ref, b_ref, out_ref):
    cid = lax.broadcasted_iota(jnp.int32, (16, 1), 0)
    nid = jnp.minimum(cid // 3, 4)
    pid = cid - (cid // 3) * 3
    eb = jnp.zeros((16, _D), jnp.float32)
    for r in range(5):
        eb = eb + jnp.where(nid == r, 1.0, 0.0) * base_ref[r : r + 1, :]
    ep = jnp.zeros((16, _D), jnp.float32)
    for r in range(3):
        ep = ep + jnp.where(pid == r, 1.0, 0.0) * pos_ref[r : r + 1, :]
    e = jnp.concatenate([eb, ep], axis=1)
    fused = (
        jnp.dot(e, w_ref[...], preferred_element_type=jnp.float32) + b_ref[...]
    )
    i = pl.program_id(0)
    q = i * _QBLK + lax.broadcasted_iota(jnp.int32, (_QBLK, 1), 0)
    lane = lax.broadcasted_iota(jnp.int32, (1, 16), 1)
    hots = []
    bands = []
    zero = jnp.zeros((16, _D), jnp.float32)
    for k in range(4):
        ck = (q >> (12 - 4 * k)) & 15
        hots.append(jnp.where(ck == lane, 1.0, 0.0))
        bands.append(
            jnp.concatenate(
                [zero] * k + [fused] + [zero] * (3 - k), axis=1
            )
        )
    onehot64 = jnp.concatenate(hots, axis=1)
    bd = jnp.concatenate(bands, axis=0)
    out_ref[...] = jnp.dot(onehot64, bd, preferred_element_type=jnp.float32)


def _build_quad_table(base_table, pos_table, W, b):
    return pl.pallas_call(
        _quad_body,
        grid=(_TROWS // _QBLK,),
        in_specs=[
            pl.BlockSpec((5, _D), lambda i: (0, 0)),
            pl.BlockSpec((3, _D), lambda i: (0, 0)),
            pl.BlockSpec((2 * _D, _D), lambda i: (0, 0)),
            pl.BlockSpec((1, _D), lambda i: (0, 0)),
        ],
        out_specs=pl.BlockSpec((_QBLK, 4 * _D), lambda i: (i, 0)),
        out_shape=jax.ShapeDtypeStruct((_TROWS, 4 * _D), jnp.float32),
    )(base_table, pos_table, W, b.reshape(1, _D))


def _sc_body(tbl_hbm, n_hbm, p_hbm, out_hbm, *scratch):
    n_v = scratch[0:2]
    p_v = scratch[2:4]
    q_v = scratch[4:6]
    rows_v = scratch[6:8]
    isem = scratch[8:10]
    gsem = scratch[10:12]
    wsem = scratch[12:14]
    wid = lax.axis_index("s") * _NC + lax.axis_index("c")
    ebase = wid * _NCHUNK * _CHUNK
    qbase = wid * _NCHUNK * _QCHUNK

    def idx_start(g, b):
        off = ebase + g * _CHUNK
        pltpu.async_copy(n_hbm.at[pl.ds(off, _CHUNK)], n_v[b], isem[b])
        pltpu.async_copy(p_hbm.at[pl.ds(off, _CHUNK)], p_v[b], isem[b])

    def idx_wait(b):
        pltpu.make_async_copy(n_hbm.at[pl.ds(0, _CHUNK)], n_v[b], isem[b]).wait()
        pltpu.make_async_copy(p_hbm.at[pl.ds(0, _CHUNK)], p_v[b], isem[b]).wait()

    def compute_q(b):
        nb, pb, qb = n_v[b], p_v[b], q_v[b]

        @pl.loop(0, _QCHUNK, step=16)
        def _pack(q0):
            acc = nb[pl.ds(q0, 16)] * 3 + pb[pl.ds(q0, 16)]
            for j in range(1, 4):
                s_ = pl.ds(q0 + _QCHUNK * j, 16)
                acc = acc * 16 + (nb[s_] * 3 + pb[s_])
            qb[pl.ds(q0, 16)] = acc

    def gathers_start(b):
        for j in range(_NG):
            s_ = pl.ds(j * _GWIN, _GWIN)
            pltpu.async_copy(
                tbl_hbm.at[q_v[b].at[s_]], rows_v[b].at[s_], gsem[b]
            )

    def gathers_wait(b):
        for j in range(_NG):
            s_ = pl.ds(j * _GWIN, _GWIN)
            pltpu.make_async_copy(
                tbl_hbm.at[q_v[b].at[s_]], rows_v[b].at[s_], gsem[b]
            ).wait()

    def write_start(g, b):
        off = qbase + g * _QCHUNK
        pltpu.async_copy(rows_v[b], out_hbm.at[pl.ds(off, _QCHUNK)], wsem[b])

    def write_wait(b):
        pltpu.make_async_copy(
            rows_v[b], out_hbm.at[pl.ds(0, _QCHUNK)], wsem[b]
        ).wait()

    def step(g, b):
        idx_wait(b)

        @pl.when(g >= 2)
        def _():
            write_wait(b)

        compute_q(b)
        gathers_start(b)

        @pl.when(g + 2 < _NCHUNK)
        def _():
            idx_start(g + 2, b)

        @pl.when(g >= 1)
        def _():
            gathers_wait(b ^ 1)
            write_start(g - 1, b ^ 1)

    idx_start(0, 0)
    idx_start(1, 1)

    @pl.loop(0, (_NCHUNK - 1) // 2)
    def _pair(gg):
        for b in range(2):
            step(gg * 2 + b, b)

    last = _NCHUNK - 1
    step(last, last & 1)
    gathers_wait(last & 1)
    write_start(last, last & 1)
    write_wait(last & 1)
    write_wait((last & 1) ^ 1)


def _sc_compiler_params():
    cp = pltpu.CompilerParams()
    if "needs_layout_passes" in pltpu.CompilerParams.__dataclass_fields__:
        cp = dataclasses.replace(cp, needs_layout_passes=False)
    return cp


@jax.jit
def _sc_lookup(tbl, n_flat, p_flat):
    mesh = plsc.VectorSubcoreMesh(core_axis_name="c", subcore_axis_name="s")
    run = pl.kernel(
        _sc_body,
        out_type=jax.ShapeDtypeStruct((_Q, 4 * _D), jnp.float32),
        mesh=mesh,
        scratch_types=(
            [pltpu.VMEM((_CHUNK,), jnp.int32) for _ in range(2)]
            + [pltpu.VMEM((_CHUNK,), jnp.int32) for _ in range(2)]
            + [pltpu.VMEM((_QCHUNK,), jnp.int32) for _ in range(2)]
            + [pltpu.VMEM((_QCHUNK, 4 * _D), jnp.float32) for _ in range(2)]
            + [pltpu.SemaphoreType.DMA for _ in range(6)]
        ),
        compiler_params=_sc_compiler_params(),
    )
    return run(tbl, n_flat, p_flat)


_XS = 4


def _xpose_body(in_ref, out_ref):
    for s_ in range(_XS):
        xt = in_ref[s_].T
        for j in range(4):
            for k in range(4):
                out_ref[s_, :, 1024 * k + 256 * j : 1024 * k + 256 * j + 256] = (
                    xt[32 * j : 32 * (j + 1), 256 * k : 256 * (k + 1)]
                )


def _to_output_layout(out_sc):
    in3 = out_sc.reshape(_S, _B // 4, 128)
    out_t = pl.pallas_call(
        _xpose_body,
        grid=(_S // _XS,),
        in_specs=[pl.BlockSpec((_XS, _B // 4, 128), lambda s: (s, 0, 0))],
        out_specs=pl.BlockSpec((_XS, _D, _B), lambda s: (s, 0, 0)),
        out_shape=jax.ShapeDtypeStruct((_S, _D, _B), jnp.float32),
    )(in3)
    return jnp.transpose(out_t, (2, 0, 1))


def kernel(nucleotides, positions, base_table, pos_table, W, b):
    tbl4 = _build_quad_table(base_table, pos_table, W, b)
    n_flat = jnp.transpose(nucleotides).reshape(_E).astype(jnp.int32)
    p_flat = jnp.transpose(positions).reshape(_E).astype(jnp.int32)
    out_sc = _sc_lookup(tbl4, n_flat, p_flat)
    return _to_output_layout(out_sc)


# speedup vs baseline: 28.2437x; 1.0789x over previous
import dataclasses

import jax
import jax.numpy as jnp
from jax import lax
from jax.experimental import pallas as pl
from jax.experimental.pallas import tpu as pltpu
from jax.experimental.pallas import tpu_sc as plsc

_B, _S, _D = 4096, 200, 32
_E = _B * _S
_Q = _E // 4
_NC, _NS = 2, 16
_NW = _NC * _NS
_CHUNK = 1024
_QCHUNK = _CHUNK // 4
_NCHUNK = _E // _CHUNK // _NW
_GWIN = 128
_NG = _QCHUNK // _GWIN
_TROWS = 16 * 16 * 16 * 16
_QBLK = 8192


def _quad_body(base_ref, pos_ref, wt_ref, b_ref, out_ref):
    cid = lax.broadcasted_iota(jnp.int32, (16, 1), 0)
    nid = jnp.minimum(cid // 3, 4)
    pid = cid - (cid // 3) * 3
    eb = jnp.zeros((16, _D), jnp.float32)
    for r in range(5):
        eb = eb + jnp.where(nid == r, 1.0, 0.0) * base_ref[r : r + 1, :]
    ep = jnp.zeros((16, _D), jnp.float32)
    for r in range(3):
        ep = ep + jnp.where(pid == r, 1.0, 0.0) * pos_ref[r : r + 1, :]
    e = jnp.concatenate([eb, ep], axis=1)
    fused = (
        lax.dot_general(
            e,
            wt_ref[...],
            (((1,), (1,)), ((), ())),
            preferred_element_type=jnp.float32,
        )
        + b_ref[...]
    )
    i = pl.program_id(0)
    q = i * _QBLK + lax.broadcasted_iota(jnp.int32, (_QBLK, 1), 0)
    lane = lax.broadcasted_iota(jnp.int32, (1, 16), 1)
    hots = []
    bands = []
    zero = jnp.zeros((16, _D), jnp.float32)
    for k in range(4):
        ck = (q >> (12 - 4 * k)) & 15
        hots.append(jnp.where(ck == lane, 1.0, 0.0))
        bands.append(
            jnp.concatenate(
                [zero] * k + [fused] + [zero] * (3 - k), axis=1
            )
        )
    onehot64 = jnp.concatenate(hots, axis=1)
    bd = jnp.concatenate(bands, axis=0)
    out_ref[...] = jnp.dot(onehot64, bd, preferred_element_type=jnp.float32)


def _build_quad_table(base_table, pos_table, W, b):
    return pl.pallas_call(
        _quad_body,
        grid=(_TROWS // _QBLK,),
        in_specs=[
            pl.BlockSpec((5, _D), lambda i: (0, 0)),
            pl.BlockSpec((3, _D), lambda i: (0, 0)),
            pl.BlockSpec((_D, 2 * _D), lambda i: (0, 0)),
            pl.BlockSpec((1, _D), lambda i: (0, 0)),
        ],
        out_specs=pl.BlockSpec((_QBLK, 4 * _D), lambda i: (i, 0)),
        out_shape=jax.ShapeDtypeStruct((_TROWS, 4 * _D), jnp.float32),
    )(base_table, pos_table, jnp.transpose(W), b.reshape(1, _D))


def _sc_body(tbl_hbm, n_hbm, p_hbm, out_hbm, *scratch):
    n_v = scratch[0:2]
    p_v = scratch[2:4]
    q_v = scratch[4:6]
    rows_v = scratch[6:8]
    isem = scratch[8:10]
    gsem = scratch[10:12]
    wsem = scratch[12:14]
    wid = lax.axis_index("s") * _NC + lax.axis_index("c")
    ebase = wid * _NCHUNK * _CHUNK
    qbase = wid * _NCHUNK * _QCHUNK

    def idx_start(g, b):
        off = ebase + g * _CHUNK
        pltpu.async_copy(n_hbm.at[pl.ds(off, _CHUNK)], n_v[b], isem[b])
        pltpu.async_copy(p_hbm.at[pl.ds(off, _CHUNK)], p_v[b], isem[b])

    def idx_wait(b):
        pltpu.make_async_copy(n_hbm.at[pl.ds(0, _CHUNK)], n_v[b], isem[b]).wait()
        pltpu.make_async_copy(p_hbm.at[pl.ds(0, _CHUNK)], p_v[b], isem[b]).wait()

    def compute_q(b):
        nb, pb, qb = n_v[b], p_v[b], q_v[b]

        @pl.loop(0, _QCHUNK, step=16)
        def _pack(q0):
            acc = nb[pl.ds(q0, 16)] * 3 + pb[pl.ds(q0, 16)]
            for j in range(1, 4):
                s_ = pl.ds(q0 + _QCHUNK * j, 16)
                acc = acc * 16 + (nb[s_] * 3 + pb[s_])
            qb[pl.ds(q0, 16)] = acc

    def gathers_start(b):
        for j in range(_NG):
            s_ = pl.ds(j * _GWIN, _GWIN)
            pltpu.async_copy(
                tbl_hbm.at[q_v[b].at[s_]], rows_v[b].at[s_], gsem[b]
            )

    def gathers_wait(b):
        for j in range(_NG):
            s_ = pl.ds(j * _GWIN, _GWIN)
            pltpu.make_async_copy(
                tbl_hbm.at[q_v[b].at[s_]], rows_v[b].at[s_], gsem[b]
            ).wait()

    def write_start(g, b):
        off = qbase + g * _QCHUNK
        pltpu.async_copy(rows_v[b], out_hbm.at[pl.ds(off, _QCHUNK)], wsem[b])

    def write_wait(b):
        pltpu.make_async_copy(
            rows_v[b], out_hbm.at[pl.ds(0, _QCHUNK)], wsem[b]
        ).wait()

    def step(g, b):
        idx_wait(b)

        @pl.when(g >= 2)
        def _():
            write_wait(b)

        compute_q(b)
        gathers_start(b)

        @pl.when(g + 2 < _NCHUNK)
        def _():
            idx_start(g + 2, b)

        @pl.when(g >= 1)
        def _():
            gathers_wait(b ^ 1)
            write_start(g - 1, b ^ 1)

    idx_start(0, 0)
    idx_start(1, 1)

    @pl.loop(0, (_NCHUNK - 1) // 2)
    def _pair(gg):
        for b in range(2):
            step(gg * 2 + b, b)

    last = _NCHUNK - 1
    step(last, last & 1)
    gathers_wait(last & 1)
    write_start(last, last & 1)
    write_wait(last & 1)
    write_wait((last & 1) ^ 1)


def _sc_compiler_params():
    cp = pltpu.CompilerParams()
    if "needs_layout_passes" in pltpu.CompilerParams.__dataclass_fields__:
        cp = dataclasses.replace(cp, needs_layout_passes=False)
    return cp


@jax.jit
def _sc_lookup(tbl, n_flat, p_flat):
    mesh = plsc.VectorSubcoreMesh(core_axis_name="c", subcore_axis_name="s")
    run = pl.kernel(
        _sc_body,
        out_type=jax.ShapeDtypeStruct((_Q, 4 * _D), jnp.float32),
        mesh=mesh,
        scratch_types=(
            [pltpu.VMEM((_CHUNK,), jnp.int32) for _ in range(2)]
            + [pltpu.VMEM((_CHUNK,), jnp.int32) for _ in range(2)]
            + [pltpu.VMEM((_QCHUNK,), jnp.int32) for _ in range(2)]
            + [pltpu.VMEM((_QCHUNK, 4 * _D), jnp.float32) for _ in range(2)]
            + [pltpu.SemaphoreType.DMA for _ in range(6)]
        ),
        compiler_params=_sc_compiler_params(),
    )
    return run(tbl, n_flat, p_flat)


_XS = 8


def _xpose_body(in_ref, out_ref):
    for s_ in range(_XS):
        xt = in_ref[s_].T
        for j in range(4):
            for k in range(4):
                out_ref[s_, :, 1024 * k + 256 * j : 1024 * k + 256 * j + 256] = (
                    xt[32 * j : 32 * (j + 1), 256 * k : 256 * (k + 1)]
                )


def _to_output_layout(out_sc):
    in3 = out_sc.reshape(_S, _B // 4, 128)
    out_t = pl.pallas_call(
        _xpose_body,
        grid=(_S // _XS,),
        in_specs=[pl.BlockSpec((_XS, _B // 4, 128), lambda s: (s, 0, 0))],
        out_specs=pl.BlockSpec((_XS, _D, _B), lambda s: (s, 0, 0)),
        out_shape=jax.ShapeDtypeStruct((_S, _D, _B), jnp.float32),
    )(in3)
    return jnp.transpose(out_t, (2, 0, 1))


def kernel(nucleotides, positions, base_table, pos_table, W, b):
    tbl4 = _build_quad_table(base_table, pos_table, W, b)
    n_flat = jnp.transpose(nucleotides).reshape(_E).astype(jnp.int32)
    p_flat = jnp.transpose(positions).reshape(_E).astype(jnp.int32)
    out_sc = _sc_lookup(tbl4, n_flat, p_flat)
    return _to_output_layout(out_sc)
